# baseline (speedup 1.0000x reference)
import jax
import jax.numpy as jnp
from jax.experimental import pallas as pl
from jax.experimental.pallas import tpu as pltpu

EMB = 128
EBLK = 2000


def _r1_stats_body(a_ref, w_ref, bee_ref, wr1_ref, o_ref):
    i = pl.program_id(0)
    nl = w_ref.shape[0]
    acc = []
    for l in range(nl):
        a = jnp.broadcast_to(a_ref[...], (EBLK, EMB))
        ee = a * w_ref[l, :][None, :] + bee_ref[l, :][None, :]
        r1 = jnp.dot(ee, wr1_ref[l], preferred_element_type=jnp.float32)
        acc.append(jnp.sum(r1, axis=0, keepdims=True))
        acc.append(jnp.sum(r1 * r1, axis=0, keepdims=True))
    part = jnp.concatenate(acc, axis=0)

    @pl.when(i == 0)
    def _():
        o_ref[...] = part

    @pl.when(i != 0)
    def _():
        o_ref[...] += part


def _r1_stats(a_col, w4, bee4, wr14):
    e = a_col.shape[0]
    nl = w4.shape[0]
    out = pl.pallas_call(
        _r1_stats_body,
        grid=(e // EBLK,),
        out_shape=jax.ShapeDtypeStruct((2 * nl, EMB), jnp.float32),
        in_specs=[
            pl.BlockSpec((EBLK, 1), lambda i: (i, 0)),
            pl.BlockSpec((nl, EMB), lambda i: (0, 0)),
            pl.BlockSpec((nl, EMB), lambda i: (0, 0)),
            pl.BlockSpec((nl, EMB, EMB), lambda i: (0, 0, 0)),
        ],
        out_specs=pl.BlockSpec((2 * nl, EMB), lambda i: (0, 0)),
    )(a_col, w4, bee4, wr14)
    return out.reshape(nl, 2, EMB)


def _edge_mlp_body(a_ref, w_ref, bee_ref, wr1_ref, al_ref, be_ref, wr2_ref,
                   br2_ref, o_ref):
    a = jnp.broadcast_to(a_ref[...], (EBLK, EMB))
    ee = a * w_ref[...] + bee_ref[...]
    r1 = jnp.dot(ee, wr1_ref[...], preferred_element_type=jnp.float32)
    z = r1 * al_ref[...] + be_ref[...]
    y = jnp.maximum(z, 0.0)
    o_ref[...] = jnp.dot(y, wr2_ref[...],
                         preferred_element_type=jnp.float32) + br2_ref[...]


def _edge_mlp(a_col, w, bee, alpha, beta, wr1, wr2, br2):
    e = a_col.shape[0]
    grid = e // EBLK
    return pl.pallas_call(
        _edge_mlp_body,
        grid=(grid,),
        out_shape=jax.ShapeDtypeStruct((e, EMB), jnp.float32),
        in_specs=[
            pl.BlockSpec((EBLK, 1), lambda i: (i, 0)),
            pl.BlockSpec((1, EMB), lambda i: (0, 0)),
            pl.BlockSpec((1, EMB), lambda i: (0, 0)),
            pl.BlockSpec((EMB, EMB), lambda i: (0, 0)),
            pl.BlockSpec((1, EMB), lambda i: (0, 0)),
            pl.BlockSpec((1, EMB), lambda i: (0, 0)),
            pl.BlockSpec((EMB, EMB), lambda i: (0, 0)),
            pl.BlockSpec((1, EMB), lambda i: (0, 0)),
        ],
        out_specs=pl.BlockSpec((EBLK, EMB), lambda i: (i, 0)),
    )(a_col, w[None, :], bee[None, :], wr1, alpha[None, :], beta[None, :],
      wr2, br2[None, :])


def _node_body(h_ref, summ_ref, inv_ref, sc_ref, wm1_ref, bm1_ref, g2_ref,
               b2_ref, wm2_ref, bm2_ref, out_ref):
    h = h_ref[...]
    agg = summ_ref[...] * inv_ref[...]
    z = sc_ref[0, 0] * h + agg
    z = jnp.dot(z, wm1_ref[...], preferred_element_type=jnp.float32) + bm1_ref[...]
    mu = jnp.mean(z, axis=0, keepdims=True)
    var = jnp.mean((z - mu) ** 2, axis=0, keepdims=True)
    z = g2_ref[...] * (z - mu) * jax.lax.rsqrt(var + 1e-5) + b2_ref[...]
    z = jnp.maximum(z, 0.0)
    z = jnp.dot(z, wm2_ref[...], preferred_element_type=jnp.float32) + bm2_ref[...]
    out_ref[...] = jnp.maximum(z, 0.0)


def _node_update(h, summ, inv, scale, p):
    n = h.shape[0]
    return pl.pallas_call(
        _node_body,
        out_shape=jax.ShapeDtypeStruct((n, EMB), jnp.float32),
        in_specs=[
            pl.BlockSpec((n, EMB), lambda: (0, 0)),
            pl.BlockSpec((n, EMB), lambda: (0, 0)),
            pl.BlockSpec((n, 1), lambda: (0, 0)),
            pl.BlockSpec(memory_space=pltpu.SMEM),
            pl.BlockSpec((EMB, 2 * EMB), lambda: (0, 0)),
            pl.BlockSpec((1, 2 * EMB), lambda: (0, 0)),
            pl.BlockSpec((1, 2 * EMB), lambda: (0, 0)),
            pl.BlockSpec((1, 2 * EMB), lambda: (0, 0)),
            pl.BlockSpec((2 * EMB, EMB), lambda: (0, 0)),
            pl.BlockSpec((1, EMB), lambda: (0, 0)),
        ],
        out_specs=pl.BlockSpec((n, EMB), lambda: (0, 0)),
    )(h, summ, inv, scale,
      p["Wm1"], p["bm1"][None, :], p["g2"][None, :], p["b2"][None, :],
      p["Wm2"], p["bm2"][None, :])


def kernel(x, edge_index, edge_attr, batch, params):
    n_nodes = x.shape[0]
    n_edges = edge_index.shape[1]
    src, dst = edge_index[0], edge_index[1]

    cnt = jax.ops.segment_sum(jnp.ones((n_edges,), jnp.float32), dst,
                              num_segments=n_nodes)
    inv = (1.0 / jnp.maximum(cnt, 1.0))[:, None]

    w4 = jnp.stack([p["Wee"][0] for p in params])
    bee4 = jnp.stack([p["bee"] for p in params])
    wr14 = jnp.stack([p["Wr1"] for p in params])
    st = _r1_stats(edge_attr, w4, bee4, wr14)
    m4 = st[:, 0, :] / n_edges
    var4 = st[:, 1, :] / n_edges - m4 * m4

    h = x
    hs = []
    for li, p in enumerate(params):
        m = m4[li] + p["br1"]
        alpha = p["g1"] / jnp.sqrt(var4[li] + 1e-5)
        beta = p["b1"] - m * alpha

        r = _edge_mlp(edge_attr, p["Wee"][0], p["bee"], alpha, beta,
                      p["Wr1"], p["Wr2"], p["br2"])
        msg = jax.nn.relu(h[src] + r)
        summ = jax.ops.segment_sum(msg, dst, num_segments=n_nodes)
        scale = jnp.reshape(1.0 + p["eps"], (1, 1))
        h = _node_update(h, summ, inv, scale, p)
        hs.append(h)
    return jnp.concatenate(hs, axis=1)


# speedup vs baseline: 2.4526x; 2.4526x over previous
import functools
import jax
import jax.numpy as jnp
from jax import lax
from jax.experimental import pallas as pl
from jax.experimental.pallas import tpu as pltpu
from jax.experimental.pallas import tpu_sc as plsc

EMB = 128
EBLK = 2000
NTILE = 32
ECH = 80
NPAD = 640 * 16


def _r1_stats_body(a_ref, w_ref, bee_ref, wr1_ref, o_ref):
    i = pl.program_id(0)
    nl = w_ref.shape[0]
    acc = []
    for l in range(nl):
        a = jnp.broadcast_to(a_ref[...], (EBLK, EMB))
        ee = a * w_ref[l, :][None, :] + bee_ref[l, :][None, :]
        r1 = jnp.dot(ee, wr1_ref[l], preferred_element_type=jnp.float32)
        acc.append(jnp.sum(r1, axis=0, keepdims=True))
        acc.append(jnp.sum(r1 * r1, axis=0, keepdims=True))
    part = jnp.concatenate(acc, axis=0)

    @pl.when(i == 0)
    def _():
        o_ref[...] = part

    @pl.when(i != 0)
    def _():
        o_ref[...] += part


def _r1_stats(a_col, w4, bee4, wr14):
    e = a_col.shape[0]
    nl = w4.shape[0]
    out = pl.pallas_call(
        _r1_stats_body,
        grid=(e // EBLK,),
        out_shape=jax.ShapeDtypeStruct((2 * nl, EMB), jnp.float32),
        in_specs=[
            pl.BlockSpec((EBLK, 1), lambda i: (i, 0)),
            pl.BlockSpec((nl, EMB), lambda i: (0, 0)),
            pl.BlockSpec((nl, EMB), lambda i: (0, 0)),
            pl.BlockSpec((nl, EMB, EMB), lambda i: (0, 0, 0)),
        ],
        out_specs=pl.BlockSpec((2 * nl, EMB), lambda i: (0, 0)),
    )(a_col, w4, bee4, wr14)
    return out.reshape(nl, 2, EMB)


def _edge_mlp_body(a_ref, w_ref, bee_ref, wr1_ref, al_ref, be_ref, wr2_ref,
                   br2_ref, o_ref):
    a = jnp.broadcast_to(a_ref[...], (EBLK, EMB))
    ee = a * w_ref[...] + bee_ref[...]
    r1 = jnp.dot(ee, wr1_ref[...], preferred_element_type=jnp.float32)
    z = r1 * al_ref[...] + be_ref[...]
    y = jnp.maximum(z, 0.0)
    o_ref[...] = jnp.dot(y, wr2_ref[...],
                         preferred_element_type=jnp.float32) + br2_ref[...]


def _edge_mlp(a_col, w, bee, alpha, beta, wr1, wr2, br2):
    e = a_col.shape[0]
    grid = e // EBLK
    return pl.pallas_call(
        _edge_mlp_body,
        grid=(grid,),
        out_shape=jax.ShapeDtypeStruct((e, EMB), jnp.float32),
        in_specs=[
            pl.BlockSpec((EBLK, 1), lambda i: (i, 0)),
            pl.BlockSpec((1, EMB), lambda i: (0, 0)),
            pl.BlockSpec((1, EMB), lambda i: (0, 0)),
            pl.BlockSpec((EMB, EMB), lambda i: (0, 0)),
            pl.BlockSpec((1, EMB), lambda i: (0, 0)),
            pl.BlockSpec((1, EMB), lambda i: (0, 0)),
            pl.BlockSpec((EMB, EMB), lambda i: (0, 0)),
            pl.BlockSpec((1, EMB), lambda i: (0, 0)),
        ],
        out_specs=pl.BlockSpec((EBLK, EMB), lambda i: (i, 0)),
    )(a_col, w[None, :], bee[None, :], wr1, alpha[None, :], beta[None, :],
      wr2, br2[None, :])


def _make_sc_edge_pass(n_nodes, n_edges):
    epw = n_edges // NTILE
    nch = epw // ECH
    mesh = plsc.VectorSubcoreMesh(core_axis_name="c", subcore_axis_name="s")

    @functools.partial(
        pl.kernel, mesh=mesh,
        out_type=jax.ShapeDtypeStruct((2, NPAD, EMB), jnp.float32),
        scratch_types=[
            pltpu.VMEM((ECH,), jnp.int32),
            pltpu.VMEM((ECH,), jnp.int32),
            pltpu.VMEM((ECH, EMB), jnp.float32),
            pltpu.VMEM((ECH, EMB), jnp.float32),
            pltpu.VMEM_SHARED((NPAD, EMB), jnp.float32),
            pltpu.SemaphoreType.DMA,
        ],
    )
    def edge_pass(h_hbm, src_hbm, dst_hbm, r_hbm, out_hbm,
                  srcv, dstv, rbuf, hbuf, acc, sem):
        cid = lax.axis_index("c")
        sid = lax.axis_index("s")
        wid = sid * 2 + cid
        ebase = wid * epw

        def zrow(i, carry):
            for cb in range(EMB // 16):
                rbuf[i, pl.ds(cb * 16, 16)] = jnp.zeros((16,), jnp.float32)
            return carry

        lax.fori_loop(0, ECH, zrow, 0)
        for t in range(16):
            @pl.when(sid == t)
            def _():
                for k in range(640 // ECH):
                    pltpu.sync_copy(rbuf, acc.at[pl.ds(t * 640 + k * ECH, ECH)])
        plsc.subcore_barrier()

        def chunk(j, carry):
            eb = ebase + j * ECH
            pltpu.sync_copy(src_hbm.at[pl.ds(eb, ECH)], srcv)
            pltpu.sync_copy(dst_hbm.at[pl.ds(eb, ECH)], dstv)
            pltpu.sync_copy(r_hbm.at[pl.ds(eb, ECH)], rbuf)
            pltpu.async_copy(h_hbm.at[srcv], hbuf, sem).wait()

            def row(i, c2):
                for cb in range(EMB // 16):
                    sl = pl.ds(cb * 16, 16)
                    rbuf[i, sl] = jnp.maximum(rbuf[i, sl] + hbuf[i, sl], 0.0)
                return c2

            lax.fori_loop(0, ECH, row, 0)
            pltpu.sync_copy(rbuf, acc.at[dstv], add=True)
            return carry

        lax.fori_loop(0, nch, chunk, 0)
        plsc.subcore_barrier()

        for t in range(16):
            @pl.when(sid == t)
            def _():
                for k in range(640 // ECH):
                    st = t * 640 + k * ECH
                    pltpu.sync_copy(acc.at[pl.ds(st, ECH)], rbuf)
                    pltpu.sync_copy(rbuf, out_hbm.at[cid, pl.ds(st, ECH)])

    return edge_pass


def _make_sc_count_pass(n_edges):
    epw = n_edges // NTILE
    nch = epw // ECH
    mesh = plsc.VectorSubcoreMesh(core_axis_name="c", subcore_axis_name="s")

    @functools.partial(
        pl.kernel, mesh=mesh,
        out_type=jax.ShapeDtypeStruct((2, NPAD, EMB), jnp.float32),
        scratch_types=[
            pltpu.VMEM((ECH,), jnp.int32),
            pltpu.VMEM((ECH, EMB), jnp.float32),
            pltpu.VMEM_SHARED((NPAD, EMB), jnp.float32),
        ],
    )
    def count_pass(dst_hbm, out_hbm, dstv, obuf, acc):
        cid = lax.axis_index("c")
        sid = lax.axis_index("s")
        wid = sid * 2 + cid
        ebase = wid * epw

        def zrow(i, carry):
            for cb in range(EMB // 16):
                obuf[i, pl.ds(cb * 16, 16)] = jnp.zeros((16,), jnp.float32)
            return carry

        lax.fori_loop(0, ECH, zrow, 0)
        for t in range(16):
            @pl.when(sid == t)
            def _():
                for k in range(640 // ECH):
                    pltpu.sync_copy(obuf, acc.at[pl.ds(t * 640 + k * ECH, ECH)])

        def onerow(i, carry):
            lane = jnp.arange(16, dtype=jnp.int32)
            obuf[i, pl.ds(0, 16)] = jnp.where(lane == 0, 1.0, 0.0)
            return carry

        lax.fori_loop(0, ECH, onerow, 0)
        plsc.subcore_barrier()

        def chunk(j, carry):
            eb = ebase + j * ECH
            pltpu.sync_copy(dst_hbm.at[pl.ds(eb, ECH)], dstv)
            pltpu.sync_copy(obuf, acc.at[dstv], add=True)
            return carry

        lax.fori_loop(0, nch, chunk, 0)
        plsc.subcore_barrier()

        for t in range(16):
            @pl.when(sid == t)
            def _():
                for k in range(640 // ECH):
                    st = t * 640 + k * ECH
                    pltpu.sync_copy(acc.at[pl.ds(st, ECH)], obuf)
                    pltpu.sync_copy(obuf, out_hbm.at[cid, pl.ds(st, ECH)])

    return count_pass


def _node_body(h_ref, s0_ref, s1_ref, inv_ref, sc_ref, wm1_ref, bm1_ref,
               g2_ref, b2_ref, wm2_ref, bm2_ref, out_ref):
    h = h_ref[...]
    agg = (s0_ref[...] + s1_ref[...]) * inv_ref[...]
    z = sc_ref[0, 0] * h + agg
    z = jnp.dot(z, wm1_ref[...], preferred_element_type=jnp.float32) + bm1_ref[...]
    mu = jnp.mean(z, axis=0, keepdims=True)
    var = jnp.mean((z - mu) ** 2, axis=0, keepdims=True)
    z = g2_ref[...] * (z - mu) * jax.lax.rsqrt(var + 1e-5) + b2_ref[...]
    z = jnp.maximum(z, 0.0)
    z = jnp.dot(z, wm2_ref[...], preferred_element_type=jnp.float32) + bm2_ref[...]
    out_ref[...] = jnp.maximum(z, 0.0)


def _node_update(h, s0, s1, inv, scale, p):
    n = h.shape[0]
    return pl.pallas_call(
        _node_body,
        out_shape=jax.ShapeDtypeStruct((n, EMB), jnp.float32),
        in_specs=[
            pl.BlockSpec((n, EMB), lambda: (0, 0)),
            pl.BlockSpec((n, EMB), lambda: (0, 0)),
            pl.BlockSpec((n, EMB), lambda: (0, 0)),
            pl.BlockSpec((n, 1), lambda: (0, 0)),
            pl.BlockSpec(memory_space=pltpu.SMEM),
            pl.BlockSpec((EMB, 2 * EMB), lambda: (0, 0)),
            pl.BlockSpec((1, 2 * EMB), lambda: (0, 0)),
            pl.BlockSpec((1, 2 * EMB), lambda: (0, 0)),
            pl.BlockSpec((1, 2 * EMB), lambda: (0, 0)),
            pl.BlockSpec((2 * EMB, EMB), lambda: (0, 0)),
            pl.BlockSpec((1, EMB), lambda: (0, 0)),
        ],
        out_specs=pl.BlockSpec((n, EMB), lambda: (0, 0)),
    )(h, s0, s1, inv, scale,
      p["Wm1"], p["bm1"][None, :], p["g2"][None, :], p["b2"][None, :],
      p["Wm2"], p["bm2"][None, :])


def kernel(x, edge_index, edge_attr, batch, params):
    n_nodes = x.shape[0]
    n_edges = edge_index.shape[1]
    src, dst = edge_index[0], edge_index[1]

    edge_pass = _make_sc_edge_pass(n_nodes, n_edges)
    count_pass = _make_sc_count_pass(n_edges)
    cnts = count_pass(dst)
    cnt = cnts[0, :n_nodes, 0] + cnts[1, :n_nodes, 0]
    inv = (1.0 / jnp.maximum(cnt, 1.0))[:, None]

    w4 = jnp.stack([p["Wee"][0] for p in params])
    bee4 = jnp.stack([p["bee"] for p in params])
    wr14 = jnp.stack([p["Wr1"] for p in params])
    st = _r1_stats(edge_attr, w4, bee4, wr14)
    m4 = st[:, 0, :] / n_edges
    var4 = st[:, 1, :] / n_edges - m4 * m4

    h = x
    hs = []
    for li, p in enumerate(params):
        m = m4[li] + p["br1"]
        alpha = p["g1"] / jnp.sqrt(var4[li] + 1e-5)
        beta = p["b1"] - m * alpha

        r = _edge_mlp(edge_attr, p["Wee"][0], p["bee"], alpha, beta,
                      p["Wr1"], p["Wr2"], p["br2"])
        parts = edge_pass(h, src, dst, r)
        scale = jnp.reshape(1.0 + p["eps"], (1, 1))
        h = _node_update(h, parts[0, :n_nodes], parts[1, :n_nodes],
                         inv, scale, p)
        hs.append(h)
    return jnp.concatenate(hs, axis=1)
